# Initial kernel scaffold; baseline (speedup 1.0000x reference)
import jax
import jax.numpy as jnp
from jax.experimental import pallas as pl


def kernel(x, emb_table, pos_table):
    raise NotImplementedError("write your pallas kernel here")


# speedup vs baseline: 1.3935x; 1.3935x over previous
import functools

import jax
import jax.numpy as jnp
from jax import lax
from jax.experimental import pallas as pl
from jax.experimental.pallas import tpu as pltpu
from jax.experimental.pallas import tpu_sc as plsc

_B = 4096
_L = 200
_H = 32
_N = _B * _L
_NC = 2
_NS = 16
_NW = _NC * _NS
_ROWS_PER_W = _N // _NW
_CH = 4
_R = _CH * _L
_NCHUNK = _ROWS_PER_W // _R


def _tok_pos_body(x_hbm, emb_hbm, pos_hbm, out_hbm, idx_v, rows_v, pos_v, sem):
    wid = lax.axis_index("s") * _NC + lax.axis_index("c")
    base = wid * _ROWS_PER_W
    pltpu.sync_copy(pos_hbm, pos_v)

    def chunk_body(c, carry):
        r0 = pl.multiple_of(base + c * _R, 32)
        pltpu.sync_copy(x_hbm.at[pl.ds(r0, _R)], idx_v)
        pltpu.async_copy(emb_hbm.at[idx_v], rows_v, sem).wait()

        def pos_body(p, carry2):
            for e in range(_CH):
                row = e * _L + p
                for half in range(_H // 16):
                    sl = pl.ds(half * 16, 16)
                    rows_v[row, sl] = rows_v[row, sl] + pos_v[p, sl]
            return carry2

        lax.fori_loop(0, _L, pos_body, 0)
        pltpu.sync_copy(rows_v, out_hbm.at[pl.ds(r0, _R)])
        return carry

    lax.fori_loop(0, _NCHUNK, chunk_body, 0)


def kernel(x, emb_table, pos_table):
    x_flat = x.reshape(_N)
    mesh = plsc.VectorSubcoreMesh(core_axis_name="c", subcore_axis_name="s")
    call = functools.partial(
        pl.kernel,
        mesh=mesh,
        compiler_params=pltpu.CompilerParams(use_tc_tiling_on_sc=False),
        out_type=jax.ShapeDtypeStruct((_N, _H), jnp.float32),
        scratch_types=[
            pltpu.VMEM((_R,), jnp.int32),
            pltpu.VMEM((_R, _H), jnp.float32),
            pltpu.VMEM((_L, _H), jnp.float32),
            pltpu.SemaphoreType.DMA,
        ],
    )(_tok_pos_body)
    out = call(x_flat, emb_table, pos_table)
    return out.reshape(_B, _L, _H)


# speedup vs baseline: 1.4630x; 1.0499x over previous
import functools

import jax
import jax.numpy as jnp
from jax import lax
from jax.experimental import pallas as pl
from jax.experimental.pallas import tpu as pltpu
from jax.experimental.pallas import tpu_sc as plsc

_B = 4096
_L = 200
_H = 32
_N = _B * _L
_NC = 2
_NS = 16
_NW = _NC * _NS
_ROWS_PER_W = _N // _NW
_CH = 4
_R = _CH * _L
_NCHUNK = _ROWS_PER_W // _R


def _pos_add(rows, pos_v):
    def pos_body(p, carry):
        for e in range(_CH):
            row = e * _L + p
            for half in range(_H // 16):
                sl = pl.ds(half * 16, 16)
                rows[row, sl] = rows[row, sl] + pos_v[p, sl]
        return carry

    lax.fori_loop(0, _L, pos_body, 0)


def _tok_pos_body(x_hbm, emb_hbm, pos_hbm, out_hbm,
                  idx0, idx1, rows0, rows1, pos_v, sg0, sg1, so0, so1):
    wid = lax.axis_index("s") * _NC + lax.axis_index("c")
    base = wid * _ROWS_PER_W
    pltpu.sync_copy(pos_hbm, pos_v)

    idx = (idx0, idx1)
    rows = (rows0, rows1)
    sg = (sg0, sg1)
    so = (so0, so1)
    gather_h = [None, None]
    out_h = [None, None]

    for g in range(_NCHUNK + 1):
        b = g % 2
        if g < _NCHUNK:
            if out_h[b] is not None:
                out_h[b].wait()
            start = base + g * _R
            pltpu.sync_copy(x_hbm.at[pl.ds(start, _R)], idx[b])
            gather_h[b] = pltpu.async_copy(emb_hbm.at[idx[b]], rows[b], sg[b])
        if g >= 1:
            pb = (g - 1) % 2
            gather_h[pb].wait()
            _pos_add(rows[pb], pos_v)
            pstart = base + (g - 1) * _R
            out_h[pb] = pltpu.async_copy(
                rows[pb], out_hbm.at[pl.ds(pstart, _R)], so[pb])

    out_h[0].wait()
    out_h[1].wait()


def kernel(x, emb_table, pos_table):
    x_flat = x.reshape(_N)
    mesh = plsc.VectorSubcoreMesh(core_axis_name="c", subcore_axis_name="s")
    call = functools.partial(
        pl.kernel,
        mesh=mesh,
        compiler_params=pltpu.CompilerParams(use_tc_tiling_on_sc=False),
        out_type=jax.ShapeDtypeStruct((_N, _H), jnp.float32),
        scratch_types=[
            pltpu.VMEM((_R,), jnp.int32),
            pltpu.VMEM((_R,), jnp.int32),
            pltpu.VMEM((_R, _H), jnp.float32),
            pltpu.VMEM((_R, _H), jnp.float32),
            pltpu.VMEM((_L, _H), jnp.float32),
            pltpu.SemaphoreType.DMA,
            pltpu.SemaphoreType.DMA,
            pltpu.SemaphoreType.DMA,
            pltpu.SemaphoreType.DMA,
        ],
    )(_tok_pos_body)
    out = call(x_flat, emb_table, pos_table)
    return out.reshape(_B, _L, _H)
